# Initial kernel scaffold; baseline (speedup 1.0000x reference)
import jax
import jax.numpy as jnp
from jax.experimental import pallas as pl


def kernel(inputs, table):
    raise NotImplementedError("write your pallas kernel here")


# speedup vs baseline: 1.1050x; 1.1050x over previous
import functools

import jax
import jax.numpy as jnp
from jax import lax
from jax.experimental import pallas as pl
from jax.experimental.pallas import tpu as pltpu
from jax.experimental.pallas import tpu_sc as plsc

_BATCH = 16384
_HIST = 50
_D = 32
_B_TOTAL = _BATCH * _HIST

_NC = 2
_NS = 16
_NW = _NC * _NS
_B_PER_W = _B_TOTAL // _NW
_CHUNK = 1280
_N_CHUNKS = _B_PER_W // _CHUNK

_mesh = plsc.VectorSubcoreMesh(core_axis_name="c", subcore_axis_name="s")


@functools.partial(
    pl.kernel,
    out_type=jax.ShapeDtypeStruct((_B_TOTAL, _D), jnp.float32),
    mesh=_mesh,
    scratch_types=[
        pltpu.VMEM((_B_PER_W,), jnp.int32),
        pltpu.VMEM((_CHUNK, _D), jnp.float32),
        pltpu.SemaphoreType.DMA,
    ],
    compiler_params=pltpu.CompilerParams(use_tc_tiling_on_sc=False),
)
def _sc_gather(idx_hbm, table_hbm, out_hbm, idx_v, rows_v, sem):
    wid = lax.axis_index("s") * _NC + lax.axis_index("c")
    base = wid * _B_PER_W
    pltpu.sync_copy(idx_hbm.at[pl.ds(base, _B_PER_W)], idx_v)

    def body(j, carry):
        off = j * _CHUNK
        pltpu.async_copy(
            table_hbm.at[idx_v.at[pl.ds(off, _CHUNK)]], rows_v, sem
        ).wait()
        pltpu.sync_copy(rows_v, out_hbm.at[pl.ds(base + off, _CHUNK)])
        return carry

    lax.fori_loop(0, _N_CHUNKS, body, 0)


def kernel(inputs, table):
    idx = inputs.reshape(_B_TOTAL).astype(jnp.int32)
    out = _sc_gather(idx, table)
    return out.reshape(_BATCH, _HIST, _D)


# speedup vs baseline: 1.1137x; 1.0079x over previous
import functools

import jax
import jax.numpy as jnp
from jax import lax
from jax.experimental import pallas as pl
from jax.experimental.pallas import tpu as pltpu
from jax.experimental.pallas import tpu_sc as plsc

_BATCH = 16384
_HIST = 50
_D = 32
_B_TOTAL = _BATCH * _HIST

_NC = 2
_NS = 16
_NW = _NC * _NS
_B_PER_W = _B_TOTAL // _NW
_CHUNK = 640
_NBUF = 4
_N_CHUNKS = _B_PER_W // _CHUNK
_N_OUTER = _N_CHUNKS // _NBUF

_mesh = plsc.VectorSubcoreMesh(core_axis_name="c", subcore_axis_name="s")


@functools.partial(
    pl.kernel,
    out_type=jax.ShapeDtypeStruct((_B_TOTAL, _D), jnp.float32),
    mesh=_mesh,
    scratch_types=[
        pltpu.VMEM((_B_PER_W,), jnp.int32),
        pltpu.VMEM((_NBUF, _CHUNK, _D), jnp.float32),
        pltpu.SemaphoreType.DMA((_NBUF,)),
        pltpu.SemaphoreType.DMA((_NBUF,)),
    ],
    compiler_params=pltpu.CompilerParams(use_tc_tiling_on_sc=False),
)
def _sc_gather(idx_hbm, table_hbm, out_hbm, idx_v, rows_v, gsem, ssem):
    wid = lax.axis_index("s") * _NC + lax.axis_index("c")
    base = wid * _B_PER_W
    pltpu.sync_copy(idx_hbm.at[pl.ds(base, _B_PER_W)], idx_v)

    def gather_desc(c, b):
        return pltpu.make_async_copy(
            table_hbm.at[idx_v.at[pl.ds(c * _CHUNK, _CHUNK)]],
            rows_v.at[b],
            gsem.at[b],
        )

    def store_desc(c, b):
        return pltpu.make_async_copy(
            rows_v.at[b],
            out_hbm.at[pl.ds(base + c * _CHUNK, _CHUNK)],
            ssem.at[b],
        )

    for b in range(_NBUF - 1):
        gather_desc(b, b).start()

    def outer(i, carry):
        t0 = i * _NBUF
        for b in range(_NBUF):
            t = t0 + b
            c_next = t + _NBUF - 1
            bn = (b - 1) % _NBUF

            def refill(t=t, c_next=c_next, bn=bn, guard_prev=(b == 0)):
                if guard_prev:
                    @pl.when(t >= 1)
                    def _():
                        store_desc(t - 1, bn).wait()
                else:
                    store_desc(t - 1, bn).wait()
                gather_desc(c_next, bn).start()

            if b == 0:
                pl.when(c_next < _N_CHUNKS)(refill)
            else:
                pl.when(c_next < _N_CHUNKS)(refill)

            gather_desc(t, b).wait()
            store_desc(t, b).start()
        return carry

    lax.fori_loop(0, _N_OUTER, outer, 0)

    for b in range(_NBUF):
        store_desc(_N_CHUNKS - _NBUF + b, b).wait()


def kernel(inputs, table):
    idx = inputs.reshape(_B_TOTAL).astype(jnp.int32)
    out = _sc_gather(idx, table)
    return out.reshape(_BATCH, _HIST, _D)


# speedup vs baseline: 1.8079x; 1.6234x over previous
import functools

import jax
import jax.numpy as jnp
from jax import lax
from jax.experimental import pallas as pl
from jax.experimental.pallas import tpu as pltpu
from jax.experimental.pallas import tpu_sc as plsc

_BATCH = 16384
_HIST = 50
_D = 32

_NC = 2
_NS = 16
_NW = _NC * _NS
_ROWS_PER_W = _BATCH // _NW
_CR = 16
_NBUF = 4
_N_CHUNKS = _ROWS_PER_W // _CR
_N_OUTER = _N_CHUNKS // _NBUF

_mesh = plsc.VectorSubcoreMesh(core_axis_name="c", subcore_axis_name="s")


@functools.partial(
    pl.kernel,
    out_type=jax.ShapeDtypeStruct((_BATCH, _HIST, _D), jnp.float32),
    mesh=_mesh,
    scratch_types=[
        pltpu.VMEM((_ROWS_PER_W, _HIST), jnp.int32),
        pltpu.VMEM((_NBUF, _CR, _HIST, _D), jnp.float32),
        pltpu.SemaphoreType.DMA((_NBUF,)),
        pltpu.SemaphoreType.DMA((_NBUF,)),
    ],
    compiler_params=pltpu.CompilerParams(use_tc_tiling_on_sc=False),
)
def _sc_gather(idx_hbm, table_hbm, out_hbm, idx_v, rows_v, gsem, ssem):
    wid = lax.axis_index("s") * _NC + lax.axis_index("c")
    base = wid * _ROWS_PER_W
    pltpu.sync_copy(idx_hbm.at[pl.ds(base, _ROWS_PER_W)], idx_v)

    def start_gather(c, b):
        for j in range(_CR):
            pltpu.make_async_copy(
                table_hbm.at[idx_v.at[c * _CR + j]],
                rows_v.at[b, j],
                gsem.at[b],
            ).start()

    def wait_gather(b):
        pltpu.make_async_copy(
            out_hbm.at[pl.ds(0, _CR)], rows_v.at[b], gsem.at[b]
        ).wait()

    def store_desc(c, b):
        return pltpu.make_async_copy(
            rows_v.at[b],
            out_hbm.at[pl.ds(base + c * _CR, _CR)],
            ssem.at[b],
        )

    for b in range(_NBUF - 1):
        start_gather(b, b)

    def outer(i, carry):
        t0 = i * _NBUF
        for b in range(_NBUF):
            t = t0 + b
            c_next = t + _NBUF - 1
            bn = (b - 1) % _NBUF

            def refill(t=t, c_next=c_next, bn=bn, guard_prev=(b == 0)):
                if guard_prev:
                    @pl.when(t >= 1)
                    def _():
                        store_desc(t - 1, bn).wait()
                else:
                    store_desc(t - 1, bn).wait()
                start_gather(c_next, bn)

            pl.when(c_next < _N_CHUNKS)(refill)

            wait_gather(b)
            store_desc(t, b).start()
        return carry

    lax.fori_loop(0, _N_OUTER, outer, 0)

    for b in range(_NBUF):
        store_desc(_N_CHUNKS - _NBUF + b, b).wait()


def kernel(inputs, table):
    return _sc_gather(inputs, table)


# speedup vs baseline: 1.8167x; 1.0048x over previous
import functools

import jax
import jax.numpy as jnp
from jax import lax
from jax.experimental import pallas as pl
from jax.experimental.pallas import tpu as pltpu
from jax.experimental.pallas import tpu_sc as plsc

_BATCH = 16384
_HIST = 50
_D = 32

_NC = 2
_NS = 16
_NW = _NC * _NS
_BPW = _BATCH // _NW
_NBUF = 5
_N_OUTER = _HIST // _NBUF

_mesh = plsc.VectorSubcoreMesh(core_axis_name="c", subcore_axis_name="s")


@functools.partial(
    pl.kernel,
    out_type=jax.ShapeDtypeStruct((_BATCH, _HIST, _D), jnp.float32),
    mesh=_mesh,
    scratch_types=[
        pltpu.VMEM((_HIST, _BPW), jnp.int32),
        pltpu.VMEM((_NBUF, _BPW, _D), jnp.float32),
        pltpu.SemaphoreType.DMA((_NBUF,)),
        pltpu.SemaphoreType.DMA((_NBUF,)),
    ],
    compiler_params=pltpu.CompilerParams(use_tc_tiling_on_sc=False),
)
def _sc_gather(idxt_hbm, table_hbm, out_hbm, idx_v, rows_v, gsem, ssem):
    wid = lax.axis_index("s") * _NC + lax.axis_index("c")
    b0 = wid * _BPW
    pltpu.sync_copy(idxt_hbm.at[:, pl.ds(b0, _BPW)], idx_v)

    def start_gather(h, b):
        for c in range(4):
            pltpu.make_async_copy(
                table_hbm.at[idx_v.at[h, pl.ds(c * 128, 128)]],
                rows_v.at[b, pl.ds(c * 128, 128)],
                gsem.at[b],
            ).start()

    def wait_gather(h, b):
        pltpu.make_async_copy(
            out_hbm.at[pl.ds(b0, _BPW), h], rows_v.at[b], gsem.at[b]
        ).wait()

    def store_desc(h, b):
        return pltpu.make_async_copy(
            rows_v.at[b],
            out_hbm.at[pl.ds(b0, _BPW), h],
            ssem.at[b],
        )

    for b in range(_NBUF - 1):
        start_gather(b, b)

    def outer(i, carry):
        t0 = i * _NBUF
        for b in range(_NBUF):
            t = t0 + b
            h_next = t + _NBUF - 1
            bn = (b - 1) % _NBUF

            def refill(t=t, h_next=h_next, bn=bn, guard_prev=(b == 0)):
                if guard_prev:
                    @pl.when(t >= 1)
                    def _():
                        store_desc(t - 1, bn).wait()
                else:
                    store_desc(t - 1, bn).wait()
                start_gather(h_next, bn)

            pl.when(h_next < _HIST)(refill)

            wait_gather(t, b)
            store_desc(t, b).start()
        return carry

    lax.fori_loop(0, _N_OUTER, outer, 0)

    for b in range(_NBUF):
        store_desc(_HIST - _NBUF + b, b).wait()


def kernel(inputs, table):
    return _sc_gather(inputs.T, table)
